# baseline (speedup 1.0000x reference)
import functools

import jax
import jax.numpy as jnp
from jax import lax
from jax.experimental import pallas as pl
from jax.experimental.pallas import tpu as pltpu
from jax.experimental.pallas import tpu_sc as plsc

B = 16384
D = 32
H = 128
PACK = 4
NC = 2
NS = 16
NW = NC * NS
BPW = B // NW
CHUNK = BPW // 2


def _gather_sc(ut_r, it_r, u_hi, i_hi):
    mesh = plsc.VectorSubcoreMesh(core_axis_name="c", subcore_axis_name="s")

    @functools.partial(
        pl.kernel,
        mesh=mesh,
        out_type=[
            jax.ShapeDtypeStruct((B, PACK * D), jnp.float32),
            jax.ShapeDtypeStruct((B, PACK * D), jnp.float32),
        ],
        scratch_types=[
            pltpu.VMEM((BPW,), jnp.int32),
            pltpu.VMEM((BPW,), jnp.int32),
            pltpu.VMEM((CHUNK, PACK * D), jnp.float32),
            pltpu.VMEM((CHUNK, PACK * D), jnp.float32),
            pltpu.SemaphoreType.DMA,
            pltpu.SemaphoreType.DMA,
            pltpu.SemaphoreType.DMA,
            pltpu.SemaphoreType.DMA,
        ],
    )
    def k(ut_hbm, it_hbm, u_hbm, i_hbm, uw_hbm, iw_hbm,
          uidx_v, iidx_v, buf0, buf1, gs0, gs1, ws0, ws1):
        wid = lax.axis_index("s") * NC + lax.axis_index("c")
        base = wid * BPW
        pltpu.sync_copy(u_hbm.at[pl.ds(base, BPW)], uidx_v)
        pltpu.sync_copy(i_hbm.at[pl.ds(base, BPW)], iidx_v)

        g0 = pltpu.async_copy(ut_hbm.at[uidx_v.at[pl.ds(0, CHUNK)]], buf0, gs0)
        g1 = pltpu.async_copy(ut_hbm.at[uidx_v.at[pl.ds(CHUNK, CHUNK)]], buf1, gs1)
        g0.wait()
        w0 = pltpu.async_copy(buf0, uw_hbm.at[pl.ds(base, CHUNK)], ws0)
        g1.wait()
        w1 = pltpu.async_copy(buf1, uw_hbm.at[pl.ds(base + CHUNK, CHUNK)], ws1)
        w0.wait()
        g2 = pltpu.async_copy(it_hbm.at[iidx_v.at[pl.ds(0, CHUNK)]], buf0, gs0)
        w1.wait()
        g3 = pltpu.async_copy(it_hbm.at[iidx_v.at[pl.ds(CHUNK, CHUNK)]], buf1, gs1)
        g2.wait()
        w2 = pltpu.async_copy(buf0, iw_hbm.at[pl.ds(base, CHUNK)], ws0)
        g3.wait()
        w3 = pltpu.async_copy(buf1, iw_hbm.at[pl.ds(base + CHUNK, CHUNK)], ws1)
        w2.wait()
        w3.wait()

    return k(ut_r, it_r, u_hi, i_hi)


def _mlp_body(uw_ref, iw_ref, u_ref, i_ref, w1u_ref, w1i_ref,
              b1_ref, w2_ref, b2_ref, o_ref):
    blk = uw_ref.shape[0]
    lane = lax.broadcasted_iota(jnp.int32, (blk, PACK * D), 1) >> 5
    xu = jnp.where(lane == (u_ref[...] & (PACK - 1)), uw_ref[...], 0.0)
    xi = jnp.where(lane == (i_ref[...] & (PACK - 1)), iw_ref[...], 0.0)
    h = jnp.dot(xu, w1u_ref[...], preferred_element_type=jnp.float32,
                precision=lax.Precision.HIGHEST)
    h = h + jnp.dot(xi, w1i_ref[...], preferred_element_type=jnp.float32,
                    precision=lax.Precision.HIGHEST)
    h = jnp.maximum(h + b1_ref[...], 0.0)
    o_ref[...] = jnp.sum(h * w2_ref[...], axis=1, keepdims=True) + b2_ref[...]


def _mlp_tc(uw, iw, user, item, W1, b1, W2, b2):
    W1u4 = jnp.tile(W1[:D], (PACK, 1))
    W1i4 = jnp.tile(W1[D:], (PACK, 1))
    b1r = b1.reshape(1, H)
    w2r = W2.reshape(1, H)
    b2s = b2.reshape(1, 1)
    u2 = user.reshape(B, 1)
    i2 = item.reshape(B, 1)

    BLK = 2048
    return pl.pallas_call(
        _mlp_body,
        grid=(B // BLK,),
        in_specs=[
            pl.BlockSpec((BLK, PACK * D), lambda i: (i, 0)),
            pl.BlockSpec((BLK, PACK * D), lambda i: (i, 0)),
            pl.BlockSpec((BLK, 1), lambda i: (i, 0)),
            pl.BlockSpec((BLK, 1), lambda i: (i, 0)),
            pl.BlockSpec((PACK * D, H), lambda i: (0, 0)),
            pl.BlockSpec((PACK * D, H), lambda i: (0, 0)),
            pl.BlockSpec((1, H), lambda i: (0, 0)),
            pl.BlockSpec((1, H), lambda i: (0, 0)),
            pl.BlockSpec((1, 1), lambda i: (0, 0)),
        ],
        out_specs=pl.BlockSpec((BLK, 1), lambda i: (i, 0)),
        out_shape=jax.ShapeDtypeStruct((B, 1), jnp.float32),
    )(uw, iw, u2, i2, W1u4, W1i4, b1r, w2r, b2s)


def kernel(user, item, user_table, item_table, W1, b1, W2, b2):
    ut_r = user_table.reshape(-1, PACK * D)
    it_r = item_table.reshape(-1, PACK * D)
    u_hi = lax.shift_right_logical(user, 2)
    i_hi = lax.shift_right_logical(item, 2)
    uw, iw = _gather_sc(ut_r, it_r, u_hi, i_hi)
    return _mlp_tc(uw, iw, user, item, W1, b1, W2, b2)
